# baseline (speedup 1.0000x reference)
import functools

import jax
import jax.numpy as jnp
from jax import lax
from jax.experimental import pallas as pl
from jax.experimental.pallas import tpu as pltpu
from jax.experimental.pallas import tpu_sc as plsc

SIZE = 1024
DIM = 64
N = 32 * 576
BLK = 512
NBLK = N // BLK

NW = 32
RPW = N // NW
CHUNK = 96
NCHUNK = RPW // CHUNK


def _tc_body(z_ref, cb_ref, cn_ref, idx_ref, perp_ref, counts_ref):
    i = pl.program_id(0)

    @pl.when(i == 0)
    def _init():
        counts_ref[...] = jnp.zeros_like(counts_ref)

    z = z_ref[...]
    z16 = z.astype(jnp.bfloat16)
    cb16 = cb_ref[...].astype(jnp.bfloat16)
    scores = lax.dot_general(z16, cb16, (((1,), (1,)), ((), ())),
                             preferred_element_type=jnp.float32)
    zn = jnp.sum(z * z, axis=1, keepdims=True)
    cost = (zn + cn_ref[...]) - 2.0 * scores
    idx = jnp.argmin(cost, axis=1).astype(jnp.int32)
    idx_ref[...] = idx

    onehot = (lax.broadcasted_iota(jnp.int32, (BLK, SIZE), 1)
              == idx[:, None]).astype(jnp.float32)
    counts_ref[...] += jnp.sum(onehot, axis=0, keepdims=True)

    @pl.when(i == NBLK - 1)
    def _fin():
        e = counts_ref[...] * (1.0 / N)
        perp_ref[0, 0] = jnp.exp(-jnp.sum(e * jnp.log(e + 1e-10)))


def _tc_argmin(z_flat, codebook, cnorm, interpret=False):
    return pl.pallas_call(
        _tc_body,
        grid=(NBLK,),
        in_specs=[
            pl.BlockSpec((BLK, DIM), lambda i: (i, 0)),
            pl.BlockSpec((SIZE, DIM), lambda i: (0, 0)),
            pl.BlockSpec((1, SIZE), lambda i: (0, 0)),
        ],
        out_specs=[
            pl.BlockSpec((BLK,), lambda i: (i,)),
            pl.BlockSpec(memory_space=pltpu.SMEM),
        ],
        out_shape=[
            jax.ShapeDtypeStruct((N,), jnp.int32),
            jax.ShapeDtypeStruct((1, 1), jnp.float32),
        ],
        scratch_shapes=[pltpu.VMEM((1, SIZE), jnp.float32)],
        compiler_params=pltpu.CompilerParams(
            dimension_semantics=("arbitrary",)),
        interpret=interpret,
    )(z_flat, codebook, cnorm)


def _sc_gather_build():
    mesh = plsc.VectorSubcoreMesh(core_axis_name="c", subcore_axis_name="s")

    @functools.partial(
        pl.kernel,
        mesh=mesh,
        out_type=jax.ShapeDtypeStruct((NW, RPW, DIM), jnp.float32),
        scratch_types=[
            pltpu.VMEM((RPW,), jnp.int32),
            pltpu.VMEM((RPW, DIM), jnp.float32),
            pltpu.SemaphoreType.DMA,
        ],
        compiler_params=pltpu.CompilerParams(use_tc_tiling_on_sc=False),
    )
    def _sc_gather(cb_hbm, idx_hbm, out_hbm, idx_v, rows_v, sem):
        wid = lax.axis_index("s") * 2 + lax.axis_index("c")
        base = wid * RPW
        pltpu.sync_copy(idx_hbm.at[pl.ds(base, RPW)], idx_v)
        copies = []
        for j in range(NCHUNK):
            copies.append(pltpu.async_copy(
                cb_hbm.at[idx_v.at[pl.ds(j * CHUNK, CHUNK)]],
                rows_v.at[pl.ds(j * CHUNK, CHUNK)],
                sem))
        for c in copies:
            c.wait()
        pltpu.sync_copy(rows_v, out_hbm.at[wid])

    return _sc_gather


def kernel(z_from_encoder, codebook, codebook_weight, flg_train):
    z = z_from_encoder
    z_flat = z.reshape(-1, DIM)
    cnorm = jnp.sum(codebook ** 2, axis=1)[None, :]
    idx, perp = _tc_argmin(z_flat, codebook, cnorm)
    z_q = _sc_gather_build()(codebook, idx)
    return (z_q, 0.0, perp[0, 0])


# speedup vs baseline: 1.0962x; 1.0962x over previous
import functools

import jax
import jax.numpy as jnp
from jax import lax
from jax.experimental import pallas as pl
from jax.experimental.pallas import tpu as pltpu
from jax.experimental.pallas import tpu_sc as plsc

SIZE = 1024
DIM = 64
N = 32 * 576
B = 32
RPB = 576

NW = 32
RPW = N // NW
CHUNK = 96
NCHUNK = RPW // CHUNK


def _tc_body(zT_ref, cb_ref, cn_ref, idx_ref, perp_ref, counts_ref):
    i = pl.program_id(0)

    @pl.when(i == 0)
    def _init():
        counts_ref[...] = jnp.zeros_like(counts_ref)

    zT = zT_ref[0]
    zT16 = zT.astype(jnp.bfloat16)
    cb16 = cb_ref[...].astype(jnp.bfloat16)
    scoresT = lax.dot_general(cb16, zT16, (((1,), (0,)), ((), ())),
                              preferred_element_type=jnp.float32)
    znT = jnp.sum(zT * zT, axis=0, keepdims=True)
    costT = (znT + cn_ref[...]) - 2.0 * scoresT
    idx = jnp.argmin(costT, axis=0).astype(jnp.int32)
    idx_ref[0, 0, :] = idx

    onehotT = (lax.broadcasted_iota(jnp.int32, (SIZE, RPB), 0)
               == idx[None, :]).astype(jnp.bfloat16)
    counts_ref[...] += lax.dot_general(
        onehotT, jnp.ones((RPB, 1), jnp.bfloat16), (((1,), (0,)), ((), ())),
        preferred_element_type=jnp.float32)

    @pl.when(i == B - 1)
    def _fin():
        e = counts_ref[...] * (1.0 / N)
        perp_ref[0, 0] = jnp.exp(-jnp.sum(e * jnp.log(e + 1e-10)))


def _tc_argmin(zT, codebook, cnorm, interpret=False):
    return pl.pallas_call(
        _tc_body,
        grid=(B,),
        in_specs=[
            pl.BlockSpec((1, DIM, RPB), lambda i: (i, 0, 0)),
            pl.BlockSpec((SIZE, DIM), lambda i: (0, 0)),
            pl.BlockSpec((SIZE, 1), lambda i: (0, 0)),
        ],
        out_specs=[
            pl.BlockSpec((1, 1, RPB), lambda i: (i, 0, 0)),
            pl.BlockSpec(memory_space=pltpu.SMEM),
        ],
        out_shape=[
            jax.ShapeDtypeStruct((B, 1, RPB), jnp.int32),
            jax.ShapeDtypeStruct((1, 1), jnp.float32),
        ],
        scratch_shapes=[pltpu.VMEM((SIZE, 1), jnp.float32)],
        compiler_params=pltpu.CompilerParams(
            dimension_semantics=("arbitrary",)),
        interpret=interpret,
    )(zT, codebook, cnorm)


def _sc_gather_build():
    mesh = plsc.VectorSubcoreMesh(core_axis_name="c", subcore_axis_name="s")

    @functools.partial(
        pl.kernel,
        mesh=mesh,
        out_type=jax.ShapeDtypeStruct((NW, RPW, DIM), jnp.float32),
        scratch_types=[
            pltpu.VMEM((RPW,), jnp.int32),
            pltpu.VMEM((RPW, DIM), jnp.float32),
            pltpu.SemaphoreType.DMA,
        ],
        compiler_params=pltpu.CompilerParams(use_tc_tiling_on_sc=False),
    )
    def _sc_gather(cb_hbm, idx_hbm, out_hbm, idx_v, rows_v, sem):
        wid = lax.axis_index("s") * 2 + lax.axis_index("c")
        pltpu.sync_copy(idx_hbm.at[wid, 0], idx_v)
        copies = []
        for j in range(NCHUNK):
            copies.append(pltpu.async_copy(
                cb_hbm.at[idx_v.at[pl.ds(j * CHUNK, CHUNK)]],
                rows_v.at[pl.ds(j * CHUNK, CHUNK)],
                sem))
        for c in copies:
            c.wait()
        pltpu.sync_copy(rows_v, out_hbm.at[wid])

    return _sc_gather


def kernel(z_from_encoder, codebook, codebook_weight, flg_train):
    z = z_from_encoder
    zT = jnp.swapaxes(z, 1, 2)
    cnorm = jnp.sum(codebook ** 2, axis=1, keepdims=True)
    idx, perp = _tc_argmin(zT, codebook, cnorm)
    z_q = _sc_gather_build()(codebook, idx)
    return (z_q, 0.0, perp[0, 0])


# speedup vs baseline: 1.2002x; 1.0948x over previous
import functools

import jax
import jax.numpy as jnp
from jax import lax
from jax.experimental import pallas as pl
from jax.experimental.pallas import tpu as pltpu
from jax.experimental.pallas import tpu_sc as plsc

SIZE = 1024
DIM = 64
N = 32 * 576
B = 32
RPB = 576

NW = 32
RPW = N // NW
CHUNK = 96
NCHUNK = RPW // CHUNK
L = 16
CROWS = SIZE // L


def _tc_body(zT_ref, cb_ref, cn_ref, idx_ref):
    zT = zT_ref[0]
    zT16 = zT.astype(jnp.bfloat16)
    cb16 = cb_ref[...].astype(jnp.bfloat16)
    scoresT = lax.dot_general(cb16, zT16, (((1,), (0,)), ((), ())),
                              preferred_element_type=jnp.float32)
    znT = jnp.sum(zT * zT, axis=0, keepdims=True)
    costT = (znT + cn_ref[...]) - 2.0 * scoresT
    idx = jnp.argmin(costT, axis=0).astype(jnp.int32)
    idx_ref[0, 0, :] = idx


def _tc_argmin(zT, codebook, cnorm, interpret=False):
    return pl.pallas_call(
        _tc_body,
        grid=(B,),
        in_specs=[
            pl.BlockSpec((1, DIM, RPB), lambda i: (i, 0, 0)),
            pl.BlockSpec((SIZE, DIM), lambda i: (0, 0)),
            pl.BlockSpec((SIZE, 1), lambda i: (0, 0)),
        ],
        out_specs=[
            pl.BlockSpec((1, 1, RPB), lambda i: (i, 0, 0)),
        ],
        out_shape=[
            jax.ShapeDtypeStruct((B, 1, RPB), jnp.int32),
        ],
        compiler_params=pltpu.CompilerParams(
            dimension_semantics=("arbitrary",)),
        interpret=interpret,
    )(zT, codebook, cnorm)


def _tc_perp_body(c_ref, perp_ref):
    c = c_ref[0] + c_ref[1]
    e = c[:, 0:1] * (1.0 / N)
    perp_ref[0, 0] = jnp.exp(-jnp.sum(e * jnp.log(e + 1e-10)))


def _tc_perplexity(counts2):
    return pl.pallas_call(
        _tc_perp_body,
        out_specs=pl.BlockSpec(memory_space=pltpu.SMEM),
        out_shape=jax.ShapeDtypeStruct((1, 1), jnp.float32),
    )(counts2)


def _sc_gather_build():
    mesh = plsc.VectorSubcoreMesh(core_axis_name="c", subcore_axis_name="s")

    @functools.partial(
        pl.kernel,
        mesh=mesh,
        out_type=(
            jax.ShapeDtypeStruct((NW, RPW, DIM), jnp.float32),
            jax.ShapeDtypeStruct((2, SIZE, L), jnp.float32),
        ),
        scratch_types=[
            pltpu.VMEM((NCHUNK, CHUNK), jnp.int32),
            pltpu.VMEM((RPW, DIM), jnp.float32),
            pltpu.VMEM((CROWS, L), jnp.float32),
            pltpu.VMEM((CHUNK, L), jnp.float32),
            pltpu.VMEM_SHARED((SIZE, L), jnp.float32),
            pltpu.SemaphoreType.DMA,
        ],
        compiler_params=pltpu.CompilerParams(use_tc_tiling_on_sc=False),
    )
    def _sc_gather(cb_hbm, idx_hbm, out_hbm, cnt_hbm,
                   idx_v, rows_v, zeros_v, ones_v, shared, sem):
        cid = lax.axis_index("c")
        sid = lax.axis_index("s")
        wid = sid * 2 + cid
        pltpu.sync_copy(idx_hbm.at[wid], idx_v)
        copies = []
        for j in range(NCHUNK):
            copies.append(pltpu.async_copy(
                cb_hbm.at[idx_v.at[j]],
                rows_v.at[pl.ds(j * CHUNK, CHUNK)],
                sem))

        def _zero(j, _):
            zeros_v[j, :] = jnp.zeros((L,), jnp.float32)
            return 0
        lax.fori_loop(0, CROWS, _zero, 0)

        def _one(j, _):
            ones_v[j, :] = jnp.ones((L,), jnp.float32)
            return 0
        lax.fori_loop(0, CHUNK, _one, 0)

        @pl.when(sid == 0)
        def _zero_shared():
            def _zs(j, _):
                pltpu.sync_copy(zeros_v, shared.at[pl.ds(j * CROWS, CROWS)])
                return 0
            lax.fori_loop(0, SIZE // CROWS, _zs, 0)
        plsc.subcore_barrier()

        for j in range(NCHUNK):
            pltpu.sync_copy(ones_v, shared.at[idx_v.at[j]], add=True)
        plsc.subcore_barrier()

        @pl.when(sid == 0)
        def _emit_counts():
            pltpu.sync_copy(shared, cnt_hbm.at[cid])

        for c in copies:
            c.wait()
        pltpu.sync_copy(rows_v, out_hbm.at[wid])

    return _sc_gather


def kernel(z_from_encoder, codebook, codebook_weight, flg_train):
    z = z_from_encoder
    zT = jnp.swapaxes(z, 1, 2)
    cnorm = jnp.sum(codebook ** 2, axis=1, keepdims=True)
    idx = _tc_argmin(zT, codebook, cnorm)[0]
    idx_sc = idx.reshape(NW, NCHUNK, CHUNK)
    z_q, counts2 = _sc_gather_build()(codebook, idx_sc)
    perp = _tc_perplexity(counts2)
    return (z_q, 0.0, perp[0, 0])
